# baseline (speedup 1.0000x reference)
import functools

import jax
import jax.numpy as jnp
from jax.experimental import pallas as pl
from jax.experimental.pallas import tpu as pltpu

DIM = 384
RANK = 96
E = 8
K = 2
H = 224
W = 224
P = H * W

C_BLK = 64
H_BLK = 8

_INV_SQRT2 = 0.7071067811865476


def _gelu_exact(x):
    return 0.5 * x * (1.0 + jax.lax.erf(x * _INV_SQRT2))


def _reduce_body(x_ref, fe_ref, pooled_ref):
    xb = x_ref[0]
    c, h, w = xb.shape
    zrow = jnp.zeros((c, 1, w), dtype=xb.dtype)
    up = jnp.concatenate([xb[:, 1:, :], zrow], axis=1)
    dn = jnp.concatenate([zrow, xb[:, :-1, :]], axis=1)
    sv = xb + up + dn
    zcol = jnp.zeros((c, h, 1), dtype=xb.dtype)
    lf = jnp.concatenate([sv[:, :, 1:], zcol], axis=2)
    rt = jnp.concatenate([zcol, sv[:, :, :-1]], axis=2)
    box = sv + lf + rt
    hp = 9.0 * xb - box
    ge = _gelu_exact(hp)
    fe_ref[...] = jnp.sum(ge, axis=1)
    pooled_ref[...] = jnp.sum(xb, axis=1)


def _router_body(fe_ref, pooled_ref, w1_ref, b1_ref, w2_ref, b2_ref,
                 gate_ref, freq_ref, idx_ref, vals_ref):
    inv = 1.0 / float(P)
    fe0 = jnp.sum(fe_ref[...], axis=1, keepdims=True) * inv
    pooled = jnp.sum(pooled_ref[...], axis=1, keepdims=True) * inv
    h1 = jnp.dot(w1_ref[...], fe0, preferred_element_type=jnp.float32)
    h1 = _gelu_exact(h1 + b1_ref[...])
    fe2 = jnp.dot(w2_ref[...], h1, preferred_element_type=jnp.float32)
    fe2 = fe2 + b2_ref[...]
    logits = (jnp.dot(gate_ref[...], pooled, preferred_element_type=jnp.float32)
              + jnp.dot(freq_ref[...], fe2, preferred_element_type=jnp.float32))
    m = jnp.max(logits, axis=0, keepdims=True)
    ex = jnp.exp(logits - m)
    s = ex / jnp.sum(ex, axis=0, keepdims=True)
    row = jax.lax.broadcasted_iota(jnp.int32, (E, 1), 0)
    v0 = jnp.max(s, axis=0, keepdims=True)
    idx0 = jnp.min(jnp.where(s == v0, row, E), axis=0, keepdims=True)
    s_masked = jnp.where(row == idx0, -jnp.inf, s)
    v1 = jnp.max(s_masked, axis=0, keepdims=True)
    idx1 = jnp.min(jnp.where(s_masked == v1, row, E), axis=0, keepdims=True)
    lane = jax.lax.broadcasted_iota(jnp.int32, (1, 128), 1)
    idx_ref[...] = jnp.where(lane == 0, idx0, jnp.where(lane == 1, idx1, 0))
    vals_ref[...] = jnp.where(lane == 0, v0, jnp.where(lane == 1, v1, 0.0))


def _main_body(idx_ref, vals_ref, x_ref, s_ref, p0a_ref, p0b_ref,
               p1a_ref, p1b_ref, p2a_ref, p2b_ref, proj_ref, out_ref,
               a_s, b_s, c_s):
    @pl.when(pl.program_id(0) == 0)
    def _init():
        a_s[...] = jnp.concatenate([p0a_ref[0], p0b_ref[0]], axis=0)
        b_s[...] = jnp.concatenate([p1a_ref[0], p1b_ref[0]], axis=0)
        c_s[...] = jnp.concatenate(
            [p2a_ref[0] * vals_ref[0], p2b_ref[0] * vals_ref[1]], axis=1)

    gs = vals_ref[0] + vals_ref[1]
    aw = a_s[...]
    bw = b_s[...]
    cw = c_s[...]
    pw = proj_ref[...]
    for j in range(H_BLK):
        xt = x_ref[0, :, j, :]
        h = jnp.dot(aw, xt, preferred_element_type=jnp.float32)
        gg = jnp.dot(bw, s_ref[0, :, j, :], preferred_element_type=jnp.float32)
        g = gg * jax.nn.sigmoid(gg)
        u = jnp.dot(cw, h * g, preferred_element_type=jnp.float32)
        out_ref[0, :, j, :] = jnp.dot(pw, u + gs * xt,
                                      preferred_element_type=jnp.float32)


@functools.partial(jax.jit, static_argnames=("interpret",))
def kernel(x, shared, mlp_w1, mlp_b1, mlp_w2, mlp_b2, gate_w, freq_gate_w,
           p0, p1, p2, proj_out_w, interpret=False):
    f32 = jnp.float32

    fe_part, pooled_part = pl.pallas_call(
        _reduce_body,
        grid=(DIM // C_BLK,),
        in_specs=[pl.BlockSpec((1, C_BLK, H, W), lambda i: (0, i, 0, 0))],
        out_specs=[pl.BlockSpec((C_BLK, W), lambda i: (i, 0)),
                   pl.BlockSpec((C_BLK, W), lambda i: (i, 0))],
        out_shape=[jax.ShapeDtypeStruct((DIM, W), f32),
                   jax.ShapeDtypeStruct((DIM, W), f32)],
        interpret=interpret,
    )(x)

    idxv, valsv = pl.pallas_call(
        _router_body,
        out_shape=[jax.ShapeDtypeStruct((1, 128), jnp.int32),
                   jax.ShapeDtypeStruct((1, 128), f32)],
        interpret=interpret,
    )(fe_part, pooled_part, mlp_w1, mlp_b1.reshape(2 * DIM, 1),
      mlp_w2, mlp_b2.reshape(DIM, 1), gate_w, freq_gate_w)

    idx = idxv[0, :K]
    vals = valsv[0, :K]

    grid_spec = pltpu.PrefetchScalarGridSpec(
        num_scalar_prefetch=2,
        grid=(H // H_BLK,),
        in_specs=[
            pl.BlockSpec((1, DIM, H_BLK, W), lambda p, i, v: (0, 0, p, 0)),
            pl.BlockSpec((1, DIM, H_BLK, W), lambda p, i, v: (0, 0, p, 0)),
            pl.BlockSpec((1, RANK, DIM), lambda p, i, v: (i[0], 0, 0)),
            pl.BlockSpec((1, RANK, DIM), lambda p, i, v: (i[1], 0, 0)),
            pl.BlockSpec((1, RANK, DIM), lambda p, i, v: (i[0], 0, 0)),
            pl.BlockSpec((1, RANK, DIM), lambda p, i, v: (i[1], 0, 0)),
            pl.BlockSpec((1, DIM, RANK), lambda p, i, v: (i[0], 0, 0)),
            pl.BlockSpec((1, DIM, RANK), lambda p, i, v: (i[1], 0, 0)),
            pl.BlockSpec((DIM, DIM), lambda p, i, v: (0, 0)),
        ],
        out_specs=pl.BlockSpec((1, DIM, H_BLK, W), lambda p, i, v: (0, 0, p, 0)),
        scratch_shapes=[
            pltpu.VMEM((K * RANK, DIM), f32),
            pltpu.VMEM((K * RANK, DIM), f32),
            pltpu.VMEM((DIM, K * RANK), f32),
        ],
    )

    return pl.pallas_call(
        _main_body,
        grid_spec=grid_spec,
        out_shape=jax.ShapeDtypeStruct((1, DIM, H, W), f32),
        interpret=interpret,
    )(idx, vals, x, shared, p0, p0, p1, p1, p2, p2, proj_out_w)


# speedup vs baseline: 1.0685x; 1.0685x over previous
import functools

import jax
import jax.numpy as jnp
from jax.experimental import pallas as pl
from jax.experimental.pallas import tpu as pltpu

DIM = 384
RANK = 96
E = 8
K = 2
H = 224
W = 224
P = H * W

C_BLK = 64
T = 1792

_INV_SQRT2 = 0.7071067811865476


def _gelu_exact(x):
    return 0.5 * x * (1.0 + jax.lax.erf(x * _INV_SQRT2))


def _reduce_body(x_ref, fe_ref, pooled_ref):
    xb = x_ref[0]
    c, h, w = xb.shape
    zrow = jnp.zeros((c, 1, w), dtype=xb.dtype)
    up = jnp.concatenate([xb[:, 1:, :], zrow], axis=1)
    dn = jnp.concatenate([zrow, xb[:, :-1, :]], axis=1)
    sv = xb + up + dn
    zcol = jnp.zeros((c, h, 1), dtype=xb.dtype)
    lf = jnp.concatenate([sv[:, :, 1:], zcol], axis=2)
    rt = jnp.concatenate([zcol, sv[:, :, :-1]], axis=2)
    box = sv + lf + rt
    hp = 9.0 * xb - box
    ge = _gelu_exact(hp)
    fe_ref[...] = jnp.sum(ge, axis=1)
    pooled_ref[...] = jnp.sum(xb, axis=1)


def _router_body(fe_ref, pooled_ref, w1_ref, b1_ref, w2_ref, b2_ref,
                 gate_ref, freq_ref, idx_ref, vals_ref):
    inv = 1.0 / float(P)
    fe0 = jnp.sum(fe_ref[...], axis=1, keepdims=True) * inv
    pooled = jnp.sum(pooled_ref[...], axis=1, keepdims=True) * inv
    h1 = jnp.dot(w1_ref[...], fe0, preferred_element_type=jnp.float32)
    h1 = _gelu_exact(h1 + b1_ref[...])
    fe2 = jnp.dot(w2_ref[...], h1, preferred_element_type=jnp.float32)
    fe2 = fe2 + b2_ref[...]
    logits = (jnp.dot(gate_ref[...], pooled, preferred_element_type=jnp.float32)
              + jnp.dot(freq_ref[...], fe2, preferred_element_type=jnp.float32))
    m = jnp.max(logits, axis=0, keepdims=True)
    ex = jnp.exp(logits - m)
    s = ex / jnp.sum(ex, axis=0, keepdims=True)
    row = jax.lax.broadcasted_iota(jnp.int32, (E, 1), 0)
    v0 = jnp.max(s, axis=0, keepdims=True)
    idx0 = jnp.min(jnp.where(s == v0, row, E), axis=0, keepdims=True)
    s_masked = jnp.where(row == idx0, -jnp.inf, s)
    v1 = jnp.max(s_masked, axis=0, keepdims=True)
    idx1 = jnp.min(jnp.where(s_masked == v1, row, E), axis=0, keepdims=True)
    lane = jax.lax.broadcasted_iota(jnp.int32, (1, 128), 1)
    idx_ref[...] = jnp.where(lane == 0, idx0, jnp.where(lane == 1, idx1, 0))
    vals_ref[...] = jnp.where(lane == 0, v0, jnp.where(lane == 1, v1, 0.0))


def _main_body(idx_ref, vals_ref, x_ref, s_ref, p0a_ref, p0b_ref,
               p1a_ref, p1b_ref, p2a_ref, p2b_ref, proj_ref, out_ref,
               a_s, b_s, c_s):
    bf16 = jnp.bfloat16

    @pl.when(pl.program_id(0) == 0)
    def _init():
        a_s[...] = jnp.concatenate([p0a_ref[0], p0b_ref[0]], axis=0)
        b_s[...] = jnp.concatenate([p1a_ref[0], p1b_ref[0]], axis=0)
        c_s[...] = jnp.concatenate(
            [p2a_ref[0].astype(jnp.float32) * vals_ref[0],
             p2b_ref[0].astype(jnp.float32) * vals_ref[1]],
            axis=1).astype(bf16)

    gs = vals_ref[0] + vals_ref[1]
    xt = x_ref[0]
    h = jnp.dot(a_s[...], xt, preferred_element_type=jnp.float32)
    gg = jnp.dot(b_s[...], s_ref[0], preferred_element_type=jnp.float32)
    g = gg * jax.nn.sigmoid(gg)
    u = jnp.dot(c_s[...], (h * g).astype(bf16),
                preferred_element_type=jnp.float32)
    r = u + gs * xt.astype(jnp.float32)
    out_ref[0] = jnp.dot(proj_ref[...], r.astype(bf16),
                         preferred_element_type=jnp.float32)


@functools.partial(jax.jit, static_argnames=("interpret",))
def kernel(x, shared, mlp_w1, mlp_b1, mlp_w2, mlp_b2, gate_w, freq_gate_w,
           p0, p1, p2, proj_out_w, interpret=False):
    f32 = jnp.float32
    bf16 = jnp.bfloat16

    fe_part, pooled_part = pl.pallas_call(
        _reduce_body,
        grid=(DIM // C_BLK,),
        in_specs=[pl.BlockSpec((1, C_BLK, H, W), lambda i: (0, i, 0, 0))],
        out_specs=[pl.BlockSpec((C_BLK, W), lambda i: (i, 0)),
                   pl.BlockSpec((C_BLK, W), lambda i: (i, 0))],
        out_shape=[jax.ShapeDtypeStruct((DIM, W), f32),
                   jax.ShapeDtypeStruct((DIM, W), f32)],
        interpret=interpret,
    )(x)

    idxv, valsv = pl.pallas_call(
        _router_body,
        out_shape=[jax.ShapeDtypeStruct((1, 128), jnp.int32),
                   jax.ShapeDtypeStruct((1, 128), f32)],
        interpret=interpret,
    )(fe_part, pooled_part, mlp_w1, mlp_b1.reshape(2 * DIM, 1),
      mlp_w2, mlp_b2.reshape(DIM, 1), gate_w, freq_gate_w)

    idx = idxv[0, :K]
    vals = valsv[0, :K]

    xb = x.reshape(1, DIM, P).astype(bf16)
    sb = shared.reshape(1, DIM, P).astype(bf16)
    p0b_ = p0.astype(bf16)
    p1b_ = p1.astype(bf16)
    projb = proj_out_w.astype(bf16)

    grid_spec = pltpu.PrefetchScalarGridSpec(
        num_scalar_prefetch=2,
        grid=(P // T,),
        in_specs=[
            pl.BlockSpec((1, DIM, T), lambda p, i, v: (0, 0, p)),
            pl.BlockSpec((1, DIM, T), lambda p, i, v: (0, 0, p)),
            pl.BlockSpec((1, RANK, DIM), lambda p, i, v: (i[0], 0, 0)),
            pl.BlockSpec((1, RANK, DIM), lambda p, i, v: (i[1], 0, 0)),
            pl.BlockSpec((1, RANK, DIM), lambda p, i, v: (i[0], 0, 0)),
            pl.BlockSpec((1, RANK, DIM), lambda p, i, v: (i[1], 0, 0)),
            pl.BlockSpec((1, DIM, RANK), lambda p, i, v: (i[0], 0, 0)),
            pl.BlockSpec((1, DIM, RANK), lambda p, i, v: (i[1], 0, 0)),
            pl.BlockSpec((DIM, DIM), lambda p, i, v: (0, 0)),
        ],
        out_specs=pl.BlockSpec((1, DIM, T), lambda p, i, v: (0, 0, p)),
        scratch_shapes=[
            pltpu.VMEM((K * RANK, DIM), bf16),
            pltpu.VMEM((K * RANK, DIM), bf16),
            pltpu.VMEM((DIM, K * RANK), bf16),
        ],
    )

    out = pl.pallas_call(
        _main_body,
        grid_spec=grid_spec,
        out_shape=jax.ShapeDtypeStruct((1, DIM, P), f32),
        interpret=interpret,
    )(idx, vals, xb, sb, p0b_, p0b_, p1b_, p1b_, p2, p2, projb)

    return out.reshape(1, DIM, H, W)


# speedup vs baseline: 1.1508x; 1.0770x over previous
import functools

import jax
import jax.numpy as jnp
from jax.experimental import pallas as pl
from jax.experimental.pallas import tpu as pltpu

DIM = 384
RANK = 96
E = 8
K = 2
H = 224
W = 224
P = H * W

C_BLK = 48
T = 3584

_INV_SQRT2 = 0.7071067811865476


def _gelu_exact(x):
    return 0.5 * x * (1.0 + jax.lax.erf(x * _INV_SQRT2))


def _reduce_router_body(x_ref, w1_ref, b1_ref, w2_ref, b2_ref,
                        gate_ref, freq_ref, idx_ref, vals_ref,
                        fe_s, pooled_s):
    i = pl.program_id(0)
    xb = x_ref[0]
    c, h, w = xb.shape
    zrow = jnp.zeros((c, 1, w), dtype=xb.dtype)
    up = jnp.concatenate([xb[:, 1:, :], zrow], axis=1)
    dn = jnp.concatenate([zrow, xb[:, :-1, :]], axis=1)
    sv = xb + up + dn
    zcol = jnp.zeros((c, h, 1), dtype=xb.dtype)
    lf = jnp.concatenate([sv[:, :, 1:], zcol], axis=2)
    rt = jnp.concatenate([zcol, sv[:, :, :-1]], axis=2)
    box = sv + lf + rt
    hp = 9.0 * xb - box
    ge = _gelu_exact(hp)
    fe_s[pl.ds(i * C_BLK, C_BLK), :] = jnp.sum(ge, axis=1)
    pooled_s[pl.ds(i * C_BLK, C_BLK), :] = jnp.sum(xb, axis=1)

    @pl.when(i == DIM // C_BLK - 1)
    def _router():
        _router_math(fe_s, pooled_s, w1_ref, b1_ref, w2_ref, b2_ref,
                     gate_ref, freq_ref, idx_ref, vals_ref)


def _router_math(fe_ref, pooled_ref, w1_ref, b1_ref, w2_ref, b2_ref,
                 gate_ref, freq_ref, idx_ref, vals_ref):
    inv = 1.0 / float(P)
    fe0 = jnp.sum(fe_ref[...], axis=1, keepdims=True) * inv
    pooled = jnp.sum(pooled_ref[...], axis=1, keepdims=True) * inv
    h1 = jnp.dot(w1_ref[...], fe0, preferred_element_type=jnp.float32)
    h1 = _gelu_exact(h1 + b1_ref[...])
    fe2 = jnp.dot(w2_ref[...], h1, preferred_element_type=jnp.float32)
    fe2 = fe2 + b2_ref[...]
    logits = (jnp.dot(gate_ref[...], pooled, preferred_element_type=jnp.float32)
              + jnp.dot(freq_ref[...], fe2, preferred_element_type=jnp.float32))
    m = jnp.max(logits, axis=0, keepdims=True)
    ex = jnp.exp(logits - m)
    s = ex / jnp.sum(ex, axis=0, keepdims=True)
    row = jax.lax.broadcasted_iota(jnp.int32, (E, 1), 0)
    v0 = jnp.max(s, axis=0, keepdims=True)
    idx0 = jnp.min(jnp.where(s == v0, row, E), axis=0, keepdims=True)
    s_masked = jnp.where(row == idx0, -jnp.inf, s)
    v1 = jnp.max(s_masked, axis=0, keepdims=True)
    idx1 = jnp.min(jnp.where(s_masked == v1, row, E), axis=0, keepdims=True)
    lane = jax.lax.broadcasted_iota(jnp.int32, (1, 128), 1)
    idx_ref[...] = jnp.where(lane == 0, idx0, jnp.where(lane == 1, idx1, 0))
    vals_ref[...] = jnp.where(lane == 0, v0, jnp.where(lane == 1, v1, 0.0))


def _main_body(idx_ref, vals_ref, x_ref, s_ref, p0a_ref, p0b_ref,
               p1a_ref, p1b_ref, p2a_ref, p2b_ref, proj_ref, out_ref,
               a_s, b_s, c_s, pj_s):
    bf16 = jnp.bfloat16

    @pl.when(pl.program_id(0) == 0)
    def _init():
        a_s[...] = jnp.concatenate([p0a_ref[0], p0b_ref[0]],
                                   axis=0).astype(bf16)
        b_s[...] = jnp.concatenate([p1a_ref[0], p1b_ref[0]],
                                   axis=0).astype(bf16)
        c_s[...] = jnp.concatenate(
            [p2a_ref[0] * vals_ref[0, 0], p2b_ref[0] * vals_ref[0, 1]],
            axis=1).astype(bf16)
        pj_s[...] = proj_ref[...].astype(bf16)

    gs = vals_ref[0, 0] + vals_ref[0, 1]
    xf = x_ref[0]
    xt = xf.astype(bf16)
    h = jnp.dot(a_s[...], xt, preferred_element_type=jnp.float32)
    gg = jnp.dot(b_s[...], s_ref[0].astype(bf16),
                 preferred_element_type=jnp.float32)
    g = gg * jax.nn.sigmoid(gg)
    u = jnp.dot(c_s[...], (h * g).astype(bf16),
                preferred_element_type=jnp.float32)
    r = u + gs * xf
    out_ref[0] = jnp.dot(pj_s[...], r.astype(bf16),
                         preferred_element_type=jnp.float32)


@functools.partial(jax.jit, static_argnames=("interpret",))
def kernel(x, shared, mlp_w1, mlp_b1, mlp_w2, mlp_b2, gate_w, freq_gate_w,
           p0, p1, p2, proj_out_w, interpret=False):
    f32 = jnp.float32
    bf16 = jnp.bfloat16

    idxv, valsv = pl.pallas_call(
        _reduce_router_body,
        grid=(DIM // C_BLK,),
        in_specs=[pl.BlockSpec((1, C_BLK, H, W), lambda i: (0, i, 0, 0)),
                  pl.BlockSpec((2 * DIM, DIM), lambda i: (0, 0)),
                  pl.BlockSpec((2 * DIM, 1), lambda i: (0, 0)),
                  pl.BlockSpec((DIM, 2 * DIM), lambda i: (0, 0)),
                  pl.BlockSpec((DIM, 1), lambda i: (0, 0)),
                  pl.BlockSpec((E, DIM), lambda i: (0, 0)),
                  pl.BlockSpec((E, DIM), lambda i: (0, 0))],
        out_specs=[pl.BlockSpec((1, 128), lambda i: (0, 0)),
                   pl.BlockSpec((1, 128), lambda i: (0, 0))],
        out_shape=[jax.ShapeDtypeStruct((1, 128), jnp.int32),
                   jax.ShapeDtypeStruct((1, 128), f32)],
        scratch_shapes=[pltpu.VMEM((DIM, W), f32),
                        pltpu.VMEM((DIM, W), f32)],
        interpret=interpret,
    )(x, mlp_w1, mlp_b1.reshape(2 * DIM, 1),
      mlp_w2, mlp_b2.reshape(DIM, 1), gate_w, freq_gate_w)

    xf = x.reshape(1, DIM, P)
    sf = shared.reshape(1, DIM, P)

    grid_spec = pltpu.PrefetchScalarGridSpec(
        num_scalar_prefetch=2,
        grid=(P // T,),
        in_specs=[
            pl.BlockSpec((1, DIM, T), lambda p, i, v: (0, 0, p)),
            pl.BlockSpec((1, DIM, T), lambda p, i, v: (0, 0, p)),
            pl.BlockSpec((1, RANK, DIM), lambda p, i, v: (i[0, 0], 0, 0)),
            pl.BlockSpec((1, RANK, DIM), lambda p, i, v: (i[0, 1], 0, 0)),
            pl.BlockSpec((1, RANK, DIM), lambda p, i, v: (i[0, 0], 0, 0)),
            pl.BlockSpec((1, RANK, DIM), lambda p, i, v: (i[0, 1], 0, 0)),
            pl.BlockSpec((1, DIM, RANK), lambda p, i, v: (i[0, 0], 0, 0)),
            pl.BlockSpec((1, DIM, RANK), lambda p, i, v: (i[0, 1], 0, 0)),
            pl.BlockSpec((DIM, DIM), lambda p, i, v: (0, 0)),
        ],
        out_specs=pl.BlockSpec((1, DIM, T), lambda p, i, v: (0, 0, p)),
        scratch_shapes=[
            pltpu.VMEM((K * RANK, DIM), bf16),
            pltpu.VMEM((K * RANK, DIM), bf16),
            pltpu.VMEM((DIM, K * RANK), bf16),
            pltpu.VMEM((DIM, DIM), bf16),
        ],
    )

    out = pl.pallas_call(
        _main_body,
        grid_spec=grid_spec,
        out_shape=jax.ShapeDtypeStruct((1, DIM, P), f32),
        interpret=interpret,
    )(idxv, valsv, xf, sf, p0, p0, p1, p1, p2, p2, proj_out_w)

    return out.reshape(1, DIM, H, W)
